# baseline (speedup 1.0000x reference)
import jax
import jax.numpy as jnp
from jax.experimental import pallas as pl
from jax.experimental.pallas import tpu as pltpu

B = 4096
NF = 26
ED = 64
IN = NF * ED
DIMS = [256, 128, 64]
E = 8
R = 16
L = 3
D = 4
KE = 2
KL = 2
DH = 64
LP = 32
SCALING = 1.0

T = 2048


def _ln(x, g, b):
    m = jnp.mean(x, axis=-1, keepdims=True)
    v = jnp.mean((x - m) ** 2, axis=-1, keepdims=True)
    return (x - m) * jax.lax.rsqrt(v + 1e-5) * g + b


def _top2_softmax(v):
    m1 = jnp.max(v, axis=-1, keepdims=True)
    neg = jnp.float32(-3.0e38)
    v2 = jnp.where(v >= m1, neg, v)
    m2 = jnp.max(v2, axis=-1, keepdims=True)
    mask = v >= m2
    e = jnp.where(mask, jnp.exp(v - m1), 0.0)
    return e / jnp.sum(e, axis=-1, keepdims=True)


_DN = (((1,), (1,)), ((), ()))


def _body(x_ref, dom_ref, de_ref, lpos_ref,
          wi1_ref, bi1_ref, gi_ref, blni_ref, wi2_ref, bi2_ref,
          wq1_ref, bq1_ref, gq_ref, blnq_ref, wq2_ref, bq2_ref,
          w1_ref, b1_ref, a1_ref, lb1_ref,
          w2_ref, b2_ref, a2_ref, lb2_ref,
          w3_ref, b3_ref, a3_ref, lb3_ref,
          wt1_ref, bt1_ref, wt2_ref, bt2_ref,
          out_ref,
          stab_ref, wa1_ref, wa2_ref, wa3_ref, bc1_ref, bc2_ref, bc3_ref,
          trow_ref):
    i = pl.program_id(0)

    @pl.when(i == 0)
    def _prep():
        lane3 = jax.lax.broadcasted_iota(jnp.int32, (D, L), 1)
        rep = (jax.lax.broadcasted_iota(jnp.int32, (E, E * R), 0)
               == jax.lax.broadcasted_iota(jnp.int32, (E, E * R), 1) // R
               ).astype(jnp.float32)
        e4 = de_ref[...]
        wi1_e, wi1_l = wi1_ref[:, :DH], wi1_ref[:, DH:]
        wq1_e, wq1_l = wq1_ref[:, :DH], wq1_ref[:, DH:]
        zi_e = jax.lax.dot_general(e4, wi1_e, _DN, preferred_element_type=jnp.float32)
        zq_e = jax.lax.dot_general(e4, wq1_e, _DN, preferred_element_type=jnp.float32)
        inter = jnp.zeros((D, L), jnp.float32)
        intra = []
        for l in range(L):
            lp = lpos_ref[l:l + 1, :]
            zi = zi_e + jax.lax.dot_general(lp, wi1_l, _DN, preferred_element_type=jnp.float32) + bi1_ref[...].reshape(1, 64)
            hi = jnp.maximum(_ln(zi, gi_ref[...].reshape(1, 64), blni_ref[...].reshape(1, 64)), 0.0)
            il = jnp.sum(hi * wi2_ref[...], axis=-1, keepdims=True) + bi2_ref[...].reshape(1, 1)
            inter = jnp.where(lane3 == l, il, inter)
            zq = zq_e + jax.lax.dot_general(lp, wq1_l, _DN, preferred_element_type=jnp.float32) + bq1_ref[...].reshape(1, 64)
            hq = jnp.maximum(_ln(zq, gq_ref[...].reshape(1, 64), blnq_ref[...].reshape(1, 64)), 0.0)
            ql = jax.lax.dot_general(hq, wq2_ref[...], _DN, preferred_element_type=jnp.float32) + bq2_ref[...].reshape(1, E)
            intra.append(_top2_softmax(ql))
        layer_w = _top2_softmax(inter)
        for l in range(L):
            lw = jnp.sum(jnp.where(lane3 == l, layer_w, 0.0), axis=-1, keepdims=True)
            gl = intra[l] * lw * SCALING
            stab_ref[l] = jnp.dot(gl, rep, preferred_element_type=jnp.float32)

        for ar, wr, war, lbr, bcr in (
                (a1_ref, w1_ref, wa1_ref, lb1_ref, bc1_ref),
                (a2_ref, w2_ref, wa2_ref, lb2_ref, bc2_ref),
                (a3_ref, w3_ref, wa3_ref, lb3_ref, bc3_ref)):
            war[:E * R, :] = ar[...].reshape(E * R, ar.shape[2]).astype(jnp.bfloat16)
            war[E * R:, :] = wr[...].astype(jnp.bfloat16)
            for e in range(E):
                bcr[e * R:(e + 1) * R, :] = jnp.swapaxes(lbr[e], 0, 1).astype(jnp.bfloat16)

        i0 = jax.lax.broadcasted_iota(jnp.int32, (8, D * 8), 0)
        i1 = jax.lax.broadcasted_iota(jnp.int32, (8, D * 8), 1)
        bt1row = jnp.zeros((1, D * 8), jnp.float32)
        wt2row = jnp.zeros((1, D * 8), jnp.float32)
        for d in range(D):
            pd = (i1 - 8 * d == i0).astype(jnp.float32)
            bt1row = bt1row + jnp.dot(bt1_ref[d:d + 1, :], pd, preferred_element_type=jnp.float32)
            wt2row = wt2row + jnp.dot(wt2_ref[d], pd, preferred_element_type=jnp.float32)
        trow_ref[0:1, :] = bt1row
        trow_ref[1:2, :] = wt2row

    dom = dom_ref[...]
    h = x_ref[...].astype(jnp.bfloat16)
    for l, (war, br, bcr) in enumerate(((wa1_ref, b1_ref, bc1_ref),
                                        (wa2_ref, b2_ref, bc2_ref),
                                        (wa3_ref, b3_ref, bc3_ref))):
        za = jax.lax.dot_general(h, war[...], _DN,
                                 preferred_element_type=jnp.float32)
        a = za[:, :E * R]
        z = za[:, E * R:] + br[...].reshape(1, br.shape[0])
        st = stab_ref[l]
        s = jnp.zeros((T, E * R), jnp.float32)
        for d in range(D):
            s = jnp.where(dom == d, st[d:d + 1, :], s)
        lora = jax.lax.dot_general((a * s).astype(jnp.bfloat16), bcr[...],
                                   (((1,), (0,)), ((), ())),
                                   preferred_element_type=jnp.float32)
        hf = jnp.maximum(z + lora, 0.0)
        h = hf.astype(jnp.bfloat16)
    h = hf

    wt1 = wt1_ref[...].reshape(D * 8, DIMS[2])
    t = jnp.maximum(
        jax.lax.dot_general(h, wt1, _DN, preferred_element_type=jnp.float32)
        + trow_ref[0:1, :], 0.0)
    blk = jax.lax.broadcasted_iota(jnp.int32, (T, D * 8), 1) // 8
    tm = jnp.where(blk == dom, t, 0.0)
    o = jnp.sum(tm * trow_ref[1:2, :], axis=-1, keepdims=True)
    ob = jnp.zeros((T, 1), jnp.float32)
    for d in range(D):
        ob = jnp.where(dom == d, bt2_ref[d:d + 1, :], ob)
    out_ref[...] = o + ob


def kernel(x, domain_id, W1, b1, W2, b2, W3, b3, loraA1, loraB1, loraA2, loraB2,
           loraA3, loraB3, dom_emb, layer_pos, Wi1, bi1, gi, bLNi, Wi2, bi2,
           Wq1, bq1, gq, bLNq, Wq2, bq2, Wt1, bt1, Wt2, bt2):
    dom2d = domain_id.astype(jnp.int32).reshape(B, 1)

    dims = [IN] + DIMS
    bf = jnp.bfloat16
    full = lambda shape: pl.BlockSpec(shape, lambda i: tuple(0 for _ in shape))
    grid = B // T
    out = pl.pallas_call(
        _body,
        grid=(grid,),
        in_specs=[
            pl.BlockSpec((T, IN), lambda i: (i, 0)),
            pl.BlockSpec((T, 1), lambda i: (i, 0)),
            full((D, DH)), full((L, LP)),
            full((64, DH + LP)), full((64,)), full((64,)), full((64,)),
            full((1, 64)), full((1,)),
            full((64, DH + LP)), full((64,)), full((64,)), full((64,)),
            full((E, 64)), full((E,)),
            full((dims[1], dims[0])), full((dims[1],)), full((E, R, dims[0])), full((E, dims[1], R)),
            full((dims[2], dims[1])), full((dims[2],)), full((E, R, dims[1])), full((E, dims[2], R)),
            full((dims[3], dims[2])), full((dims[3],)), full((E, R, dims[2])), full((E, dims[3], R)),
            full((D, 8, dims[3])), full((D, 8)),
            full((D, 1, 8)), full((D, 1)),
        ],
        out_specs=pl.BlockSpec((T, 1), lambda i: (i, 0)),
        out_shape=jax.ShapeDtypeStruct((B, 1), jnp.float32),
        scratch_shapes=[
            pltpu.VMEM((L, D, E * R), jnp.float32),
            pltpu.VMEM((E * R + dims[1], dims[0]), bf),
            pltpu.VMEM((E * R + dims[2], dims[1]), bf),
            pltpu.VMEM((E * R + dims[3], dims[2]), bf),
            pltpu.VMEM((E * R, dims[1]), bf),
            pltpu.VMEM((E * R, dims[2]), bf),
            pltpu.VMEM((E * R, dims[3]), bf),
            pltpu.VMEM((2, D * 8), jnp.float32),
        ],
    )(x, dom2d, dom_emb, layer_pos,
      Wi1, bi1, gi, bLNi,
      Wi2, bi2,
      Wq1, bq1, gq, bLNq,
      Wq2, bq2,
      W1, b1, loraA1, loraB1,
      W2, b2, loraA2, loraB2,
      W3, b3, loraA3, loraB3,
      Wt1, bt1, Wt2, bt2)
    return out


# speedup vs baseline: 1.0502x; 1.0502x over previous
import jax
import jax.numpy as jnp
from jax.experimental import pallas as pl
from jax.experimental.pallas import tpu as pltpu

B = 4096
NF = 26
ED = 64
IN = NF * ED
DIMS = [256, 128, 64]
E = 8
R = 16
L = 3
D = 4
KE = 2
KL = 2
DH = 64
LP = 32
SCALING = 1.0

T = 1024


def _ln(x, g, b):
    m = jnp.mean(x, axis=-1, keepdims=True)
    v = jnp.mean((x - m) ** 2, axis=-1, keepdims=True)
    return (x - m) * jax.lax.rsqrt(v + 1e-5) * g + b


def _top2_softmax(v):
    m1 = jnp.max(v, axis=-1, keepdims=True)
    neg = jnp.float32(-3.0e38)
    v2 = jnp.where(v >= m1, neg, v)
    m2 = jnp.max(v2, axis=-1, keepdims=True)
    mask = v >= m2
    e = jnp.where(mask, jnp.exp(v - m1), 0.0)
    return e / jnp.sum(e, axis=-1, keepdims=True)


_DN = (((1,), (1,)), ((), ()))


def _body(x_ref, dom_ref, de_ref, lpos_ref,
          wi1_ref, bi1_ref, gi_ref, blni_ref, wi2_ref, bi2_ref,
          wq1_ref, bq1_ref, gq_ref, blnq_ref, wq2_ref, bq2_ref,
          w1_ref, b1_ref, a1_ref, lb1_ref,
          w2_ref, b2_ref, a2_ref, lb2_ref,
          w3_ref, b3_ref, a3_ref, lb3_ref,
          wt1_ref, bt1_ref, wt2_ref, bt2_ref,
          out_ref,
          stab_ref, wa1_ref, wa2_ref, wa3_ref, bc1_ref, bc2_ref, bc3_ref,
          trow_ref):
    i = pl.program_id(0)

    @pl.when(i == 0)
    def _prep():
        lane3 = jax.lax.broadcasted_iota(jnp.int32, (D, L), 1)
        rep = (jax.lax.broadcasted_iota(jnp.int32, (E, E * R), 0)
               == jax.lax.broadcasted_iota(jnp.int32, (E, E * R), 1) // R
               ).astype(jnp.float32)
        e4 = de_ref[...]
        wi1_e, wi1_l = wi1_ref[:, :DH], wi1_ref[:, DH:]
        wq1_e, wq1_l = wq1_ref[:, :DH], wq1_ref[:, DH:]
        zi_e = jax.lax.dot_general(e4, wi1_e, _DN, preferred_element_type=jnp.float32)
        zq_e = jax.lax.dot_general(e4, wq1_e, _DN, preferred_element_type=jnp.float32)
        inter = jnp.zeros((D, L), jnp.float32)
        intra = []
        for l in range(L):
            lp = lpos_ref[l:l + 1, :]
            zi = zi_e + jax.lax.dot_general(lp, wi1_l, _DN, preferred_element_type=jnp.float32) + bi1_ref[...].reshape(1, 64)
            hi = jnp.maximum(_ln(zi, gi_ref[...].reshape(1, 64), blni_ref[...].reshape(1, 64)), 0.0)
            il = jnp.sum(hi * wi2_ref[...], axis=-1, keepdims=True) + bi2_ref[...].reshape(1, 1)
            inter = jnp.where(lane3 == l, il, inter)
            zq = zq_e + jax.lax.dot_general(lp, wq1_l, _DN, preferred_element_type=jnp.float32) + bq1_ref[...].reshape(1, 64)
            hq = jnp.maximum(_ln(zq, gq_ref[...].reshape(1, 64), blnq_ref[...].reshape(1, 64)), 0.0)
            ql = jax.lax.dot_general(hq, wq2_ref[...], _DN, preferred_element_type=jnp.float32) + bq2_ref[...].reshape(1, E)
            intra.append(_top2_softmax(ql))
        layer_w = _top2_softmax(inter)
        for l in range(L):
            lw = jnp.sum(jnp.where(lane3 == l, layer_w, 0.0), axis=-1, keepdims=True)
            gl = intra[l] * lw * SCALING
            stab_ref[l] = jnp.dot(gl, rep, preferred_element_type=jnp.float32)

        for ar, wr, war, lbr, bcr in (
                (a1_ref, w1_ref, wa1_ref, lb1_ref, bc1_ref),
                (a2_ref, w2_ref, wa2_ref, lb2_ref, bc2_ref),
                (a3_ref, w3_ref, wa3_ref, lb3_ref, bc3_ref)):
            war[:E * R, :] = ar[...].reshape(E * R, ar.shape[2]).astype(jnp.bfloat16)
            war[E * R:, :] = wr[...].astype(jnp.bfloat16)
            for e in range(E):
                bcr[e * R:(e + 1) * R, :] = jnp.swapaxes(lbr[e], 0, 1).astype(jnp.bfloat16)

        i0 = jax.lax.broadcasted_iota(jnp.int32, (8, D * 8), 0)
        i1 = jax.lax.broadcasted_iota(jnp.int32, (8, D * 8), 1)
        bt1row = jnp.zeros((1, D * 8), jnp.float32)
        wt2row = jnp.zeros((1, D * 8), jnp.float32)
        for d in range(D):
            pd = (i1 - 8 * d == i0).astype(jnp.float32)
            bt1row = bt1row + jnp.dot(bt1_ref[d:d + 1, :], pd, preferred_element_type=jnp.float32)
            wt2row = wt2row + jnp.dot(wt2_ref[d], pd, preferred_element_type=jnp.float32)
        trow_ref[0:1, :] = bt1row
        trow_ref[1:2, :] = wt2row

    dom = dom_ref[...]
    h = x_ref[...].astype(jnp.bfloat16)
    for l, (war, br, bcr) in enumerate(((wa1_ref, b1_ref, bc1_ref),
                                        (wa2_ref, b2_ref, bc2_ref),
                                        (wa3_ref, b3_ref, bc3_ref))):
        za = jax.lax.dot_general(h, war[...], _DN,
                                 preferred_element_type=jnp.float32)
        a = za[:, :E * R]
        z = za[:, E * R:] + br[...].reshape(1, br.shape[0])
        st = stab_ref[l]
        s = jnp.zeros((T, E * R), jnp.float32)
        for d in range(D):
            s = jnp.where(dom == d, st[d:d + 1, :], s)
        lora = jax.lax.dot_general((a * s).astype(jnp.bfloat16), bcr[...],
                                   (((1,), (0,)), ((), ())),
                                   preferred_element_type=jnp.float32)
        hf = jnp.maximum(z + lora, 0.0)
        h = hf.astype(jnp.bfloat16)
    h = hf

    wt1 = wt1_ref[...].reshape(D * 8, DIMS[2])
    t = jnp.maximum(
        jax.lax.dot_general(h, wt1, _DN, preferred_element_type=jnp.float32)
        + trow_ref[0:1, :], 0.0)
    blk = jax.lax.broadcasted_iota(jnp.int32, (T, D * 8), 1) // 8
    tm = jnp.where(blk == dom, t, 0.0)
    o = jnp.sum(tm * trow_ref[1:2, :], axis=-1, keepdims=True)
    ob = jnp.zeros((T, 1), jnp.float32)
    for d in range(D):
        ob = jnp.where(dom == d, bt2_ref[d:d + 1, :], ob)
    out_ref[...] = o + ob


def kernel(x, domain_id, W1, b1, W2, b2, W3, b3, loraA1, loraB1, loraA2, loraB2,
           loraA3, loraB3, dom_emb, layer_pos, Wi1, bi1, gi, bLNi, Wi2, bi2,
           Wq1, bq1, gq, bLNq, Wq2, bq2, Wt1, bt1, Wt2, bt2):
    dom2d = domain_id.astype(jnp.int32).reshape(B, 1)

    dims = [IN] + DIMS
    bf = jnp.bfloat16
    full = lambda shape: pl.BlockSpec(shape, lambda i: tuple(0 for _ in shape))
    grid = B // T
    out = pl.pallas_call(
        _body,
        grid=(grid,),
        in_specs=[
            pl.BlockSpec((T, IN), lambda i: (i, 0)),
            pl.BlockSpec((T, 1), lambda i: (i, 0)),
            full((D, DH)), full((L, LP)),
            full((64, DH + LP)), full((64,)), full((64,)), full((64,)),
            full((1, 64)), full((1,)),
            full((64, DH + LP)), full((64,)), full((64,)), full((64,)),
            full((E, 64)), full((E,)),
            full((dims[1], dims[0])), full((dims[1],)), full((E, R, dims[0])), full((E, dims[1], R)),
            full((dims[2], dims[1])), full((dims[2],)), full((E, R, dims[1])), full((E, dims[2], R)),
            full((dims[3], dims[2])), full((dims[3],)), full((E, R, dims[2])), full((E, dims[3], R)),
            full((D, 8, dims[3])), full((D, 8)),
            full((D, 1, 8)), full((D, 1)),
        ],
        out_specs=pl.BlockSpec((T, 1), lambda i: (i, 0)),
        out_shape=jax.ShapeDtypeStruct((B, 1), jnp.float32),
        scratch_shapes=[
            pltpu.VMEM((L, D, E * R), jnp.float32),
            pltpu.VMEM((E * R + dims[1], dims[0]), bf),
            pltpu.VMEM((E * R + dims[2], dims[1]), bf),
            pltpu.VMEM((E * R + dims[3], dims[2]), bf),
            pltpu.VMEM((E * R, dims[1]), bf),
            pltpu.VMEM((E * R, dims[2]), bf),
            pltpu.VMEM((E * R, dims[3]), bf),
            pltpu.VMEM((2, D * 8), jnp.float32),
        ],
    )(x, dom2d, dom_emb, layer_pos,
      Wi1, bi1, gi, bLNi,
      Wi2, bi2,
      Wq1, bq1, gq, bLNq,
      Wq2, bq2,
      W1, b1, loraA1, loraB1,
      W2, b2, loraA2, loraB2,
      W3, b3, loraA3, loraB3,
      Wt1, bt1, Wt2, bt2)
    return out
